# Initial kernel scaffold; baseline (speedup 1.0000x reference)
import jax
import jax.numpy as jnp
from jax.experimental import pallas as pl


def kernel(input, embedding_matrix):
    raise NotImplementedError("write your pallas kernel here")


# speedup vs baseline: 1.5658x; 1.5658x over previous
import functools

import jax
import jax.numpy as jnp
from jax import lax
from jax.experimental import pallas as pl
from jax.experimental.pallas import tpu as pltpu
from jax.experimental.pallas import tpu_sc as plsc

D = 32
NC = 2
NS = 16
NW = NC * NS
GROUP = 128
CH = 8
RPC = CH * GROUP


@functools.partial(jax.jit, static_argnames=())
def _sc_embedding_gather(idx3, table):
    nw, g, _ = idx3.shape
    b_per_w = g * GROUP
    nch = g // CH
    btot = nw * b_per_w

    mesh = plsc.VectorSubcoreMesh(core_axis_name="c", subcore_axis_name="s")

    @functools.partial(
        pl.kernel,
        out_type=jax.ShapeDtypeStruct((btot, D), jnp.float32),
        mesh=mesh,
        compiler_params=pltpu.CompilerParams(use_tc_tiling_on_sc=False),
        scratch_types=[
            pltpu.VMEM((g, GROUP), jnp.int32),
            pltpu.VMEM((2, RPC, D), jnp.float32),
            pltpu.SemaphoreType.DMA,
            pltpu.SemaphoreType.DMA,
        ],
    )
    def k(table_hbm, idx_hbm, out_hbm, idx_v, bufs, sem_g, sem_w):
        wid = lax.axis_index("s") * NC + lax.axis_index("c")
        row0 = wid * b_per_w
        pltpu.sync_copy(idx_hbm.at[wid], idx_v)

        def wait_write_one():
            pltpu.make_async_copy(
                bufs.at[0], out_hbm.at[pl.ds(0, RPC)], sem_w
            ).wait()

        def body(c, carry):
            p = lax.rem(c, 2)
            buf = bufs.at[p]

            @pl.when(c >= 2)
            def _():
                wait_write_one()

            cps = []
            for b in range(CH):
                cps.append(
                    pltpu.async_copy(
                        table_hbm.at[idx_v.at[c * CH + b]],
                        buf.at[pl.ds(b * GROUP, GROUP)],
                        sem_g,
                    )
                )
            for cp in cps:
                cp.wait()

            pltpu.async_copy(
                buf, out_hbm.at[pl.ds(row0 + c * RPC, RPC)], sem_w
            )
            return carry

        lax.fori_loop(0, nch, body, 0)
        wait_write_one()
        wait_write_one()

    return k(table, idx3)


def kernel(input, embedding_matrix):
    batch, n_fields = input.shape
    btot = batch * n_fields
    b_per_w = btot // NW
    g = b_per_w // GROUP
    idx3 = jnp.reshape(input.astype(jnp.int32), (NW, g, GROUP))
    out = _sc_embedding_gather(idx3, embedding_matrix)
    return out.reshape(batch, n_fields, D)
